# baseline (speedup 1.0000x reference)
import functools

import jax
import jax.numpy as jnp
from jax import lax
from jax.experimental import pallas as pl
from jax.experimental.pallas import tpu as pltpu
from jax.experimental.pallas import tpu_sc as plsc

MDIM = 64
NV = MDIM // 16
CELLS_PER_CHUNK = 400
NSPLIT = 2


def _sc_lookup(ids_flat, table, Rs, C):
    ncells = Rs * C
    info = plsc.get_sparse_core_info()
    nc, ns = info.num_cores, info.num_subcores
    nsub = nc * ns
    cells_per_w = ncells // nsub
    n_chunks = cells_per_w // CELLS_PER_CHUNK
    assert ncells % nsub == 0 and cells_per_w % CELLS_PER_CHUNK == 0
    assert n_chunks % 2 == 0

    mesh = plsc.VectorSubcoreMesh(core_axis_name="c", subcore_axis_name="s")

    @functools.partial(
        pl.kernel,
        out_type=jax.ShapeDtypeStruct((Rs, C, MDIM), jnp.float32),
        mesh=mesh,
        scratch_types=[
            pltpu.VMEM((2, MDIM), jnp.float32),
            pltpu.VMEM((CELLS_PER_CHUNK,), jnp.int32),
            [pltpu.VMEM((CELLS_PER_CHUNK, MDIM), jnp.float32) for _ in range(2)],
            [pltpu.SemaphoreType.DMA for _ in range(2)],
        ],
    )
    def lookup(ids_hbm, table_hbm, out_hbm, table_v, ids_v, rows, ssem):
        wid = lax.axis_index("s") * nc + lax.axis_index("c")
        cbase = wid * cells_per_w

        out_flat = out_hbm.reshape(Rs * C, MDIM)

        pltpu.sync_copy(table_hbm, table_v)
        t0 = [table_v[0, pl.ds(m * 16, 16)] for m in range(NV)]
        t1 = [table_v[1, pl.ds(m * 16, 16)] for m in range(NV)]

        def expand(g, b):
            pltpu.sync_copy(
                ids_hbm.at[pl.ds(cbase + g * CELLS_PER_CHUNK, CELLS_PER_CHUNK)],
                ids_v,
            )

            def block_body(blk, carry):
                iv = ids_v[pl.ds(blk * 16, 16)]
                cell0 = blk * 16
                for lane in range(16):
                    sel = iv[lane] == 1
                    cell = cell0 + lane
                    for m in range(NV):
                        rows[b][cell, pl.ds(m * 16, 16)] = jnp.where(
                            sel, t1[m], t0[m]
                        )
                return carry

            lax.fori_loop(0, CELLS_PER_CHUNK // 16, block_body, 0)

        def store_descr(g, b):
            return pltpu.make_async_copy(
                rows[b],
                out_flat.at[pl.ds(cbase + g * CELLS_PER_CHUNK, CELLS_PER_CHUNK)],
                ssem[b],
            )

        expand(0, 0)
        store_descr(0, 0).start()
        expand(1, 1)
        store_descr(1, 1).start()

        def body(it, carry):
            g0 = 2 * it
            for b in range(2):
                g = g0 + b
                store_descr(g, b).wait()
                expand(g, b)
                store_descr(g, b).start()
            return carry

        lax.fori_loop(1, n_chunks // 2, body, 0)

        store_descr(0, 0).wait()
        store_descr(0, 1).wait()

    return lookup(ids_flat, table)


def kernel(speaker_id, table):
    R, C = speaker_id.shape
    assert R % NSPLIT == 0
    Rs = R // NSPLIT
    ids = speaker_id.reshape(NSPLIT, Rs * C).astype(jnp.int32)
    parts = [_sc_lookup(ids[s], table, Rs, C) for s in range(NSPLIT)]
    return jnp.concatenate(parts, axis=0)


# speedup vs baseline: 1.3668x; 1.3668x over previous
import functools

import jax
import jax.numpy as jnp
from jax import lax
from jax.experimental import pallas as pl
from jax.experimental.pallas import tpu as pltpu
from jax.experimental.pallas import tpu_sc as plsc

MDIM = 64
NV = MDIM // 16
CELLS_PER_CHUNK = 400
NSPLIT = 1


def _sc_lookup(ids_flat, table, Rs, C):
    ncells = Rs * C
    info = plsc.get_sparse_core_info()
    nc, ns = info.num_cores, info.num_subcores
    nsub = nc * ns
    cells_per_w = ncells // nsub
    n_chunks = cells_per_w // CELLS_PER_CHUNK
    assert ncells % nsub == 0 and cells_per_w % CELLS_PER_CHUNK == 0
    assert n_chunks % 2 == 0

    mesh = plsc.VectorSubcoreMesh(core_axis_name="c", subcore_axis_name="s")

    @functools.partial(
        pl.kernel,
        out_type=jax.ShapeDtypeStruct((Rs, C, MDIM), jnp.float32),
        mesh=mesh,
        scratch_types=[
            pltpu.VMEM((2, MDIM), jnp.float32),
            pltpu.VMEM((CELLS_PER_CHUNK,), jnp.int32),
            [pltpu.VMEM((CELLS_PER_CHUNK, MDIM), jnp.float32) for _ in range(2)],
            [pltpu.SemaphoreType.DMA for _ in range(2)],
        ],
    )
    def lookup(ids_hbm, table_hbm, out_hbm, table_v, ids_v, rows, ssem):
        wid = lax.axis_index("s") * nc + lax.axis_index("c")
        cbase = wid * cells_per_w

        out_flat = out_hbm.reshape(Rs * C, MDIM)

        pltpu.sync_copy(table_hbm, table_v)
        t0 = [table_v[0, pl.ds(m * 16, 16)] for m in range(NV)]
        t1 = [table_v[1, pl.ds(m * 16, 16)] for m in range(NV)]

        def expand(g, b):
            pltpu.sync_copy(
                ids_hbm.at[pl.ds(cbase + g * CELLS_PER_CHUNK, CELLS_PER_CHUNK)],
                ids_v,
            )

            def block_body(blk, carry):
                iv = ids_v[pl.ds(blk * 16, 16)]
                cell0 = blk * 16
                for lane in range(16):
                    sel = iv[lane] == 1
                    cell = cell0 + lane
                    for m in range(NV):
                        rows[b][cell, pl.ds(m * 16, 16)] = jnp.where(
                            sel, t1[m], t0[m]
                        )
                return carry

            lax.fori_loop(0, CELLS_PER_CHUNK // 16, block_body, 0)

        def store_descr(g, b):
            return pltpu.make_async_copy(
                rows[b],
                out_flat.at[pl.ds(cbase + g * CELLS_PER_CHUNK, CELLS_PER_CHUNK)],
                ssem[b],
            )

        expand(0, 0)
        store_descr(0, 0).start()
        expand(1, 1)
        store_descr(1, 1).start()

        def body(it, carry):
            g0 = 2 * it
            for b in range(2):
                g = g0 + b
                store_descr(g, b).wait()
                expand(g, b)
                store_descr(g, b).start()
            return carry

        lax.fori_loop(1, n_chunks // 2, body, 0)

        store_descr(0, 0).wait()
        store_descr(0, 1).wait()

    return lookup(ids_flat, table)


def kernel(speaker_id, table):
    R, C = speaker_id.shape
    ids = speaker_id.reshape(R * C).astype(jnp.int32)
    return _sc_lookup(ids, table, R, C)
